# baseline (speedup 1.0000x reference)
import functools

import jax
import jax.numpy as jnp
from jax import lax
from jax.experimental import pallas as pl
from jax.experimental.pallas import tpu as pltpu
from jax.experimental.pallas import tpu_sc as plsc

_B = 16384
_D = 64
_NC = 2
_NS = 16
_NW = _NC * _NS
_BPW = _B // _NW
_AVG = 3.5
_L = 16
_HALF = _BPW // 2


def _body(xt_hbm, user_hbm, item_hbm, w_hbm, bias16_hbm, ubias_hbm, ibias_hbm,
          out_hbm, uidx_v, iidx_v, ub_v, ib_v, x_v, w_v, b16_v, out_v,
          gsem, xsem0, xsem1):
    wid = lax.axis_index("s") * _NC + lax.axis_index("c")
    base = wid * _BPW

    x0 = pltpu.async_copy(xt_hbm.at[:, pl.ds(base, _HALF)],
                          x_v.at[:, pl.ds(0, _HALF)], xsem0)

    pltpu.sync_copy(user_hbm.at[pl.ds(base, _BPW)], uidx_v)
    pltpu.sync_copy(item_hbm.at[pl.ds(base, _BPW)], iidx_v)
    ug = pltpu.async_copy(ubias_hbm.at[uidx_v], ub_v, gsem)
    ig = pltpu.async_copy(ibias_hbm.at[iidx_v], ib_v, gsem)

    x1 = pltpu.async_copy(xt_hbm.at[:, pl.ds(base + _HALF, _HALF)],
                          x_v.at[:, pl.ds(_HALF, _HALF)], xsem1)

    pltpu.sync_copy(w_hbm, w_v)
    pltpu.sync_copy(bias16_hbm, b16_v)

    wv = [w_v[pl.ds(k * _L, _L)] for k in range(_D // _L)]
    bconst = b16_v[pl.ds(0, _L)]

    ug.wait()
    ig.wait()

    nacc = 8

    def group(g):
        c = g * _L
        accs = [None] * nacc
        for j in range(_D):
            term = x_v[j, pl.ds(c, _L)] * wv[j // _L][j % _L]
            k = j % nacc
            accs[k] = term if accs[k] is None else accs[k] + term
        accs[0] = accs[0] + (bconst + ub_v[pl.ds(c, _L)] + ib_v[pl.ds(c, _L)])
        while len(accs) > 1:
            accs = [a + b for a, b in zip(accs[::2], accs[1::2])]
        out_v[pl.ds(c, _L)] = accs[0]

    x0.wait()
    plsc.parallel_loop(0, _HALF // _L, 1, unroll=2)(group)
    x1.wait()
    plsc.parallel_loop(_HALF // _L, _BPW // _L, 1, unroll=2)(group)

    pltpu.sync_copy(out_v, out_hbm.at[pl.ds(base, _BPW)])


@jax.jit
def _run(xt, user, item, w_flat, bias16, ubias_flat, ibias_flat):
    mesh = plsc.VectorSubcoreMesh(core_axis_name="c", subcore_axis_name="s")
    f = functools.partial(
        pl.kernel,
        out_type=jax.ShapeDtypeStruct((_B,), jnp.float32),
        mesh=mesh,
        scratch_types=[
            pltpu.VMEM((_BPW,), jnp.int32),
            pltpu.VMEM((_BPW,), jnp.int32),
            pltpu.VMEM((_BPW,), jnp.float32),
            pltpu.VMEM((_BPW,), jnp.float32),
            pltpu.VMEM((_D, _BPW), jnp.float32),
            pltpu.VMEM((_D,), jnp.float32),
            pltpu.VMEM((_L,), jnp.float32),
            pltpu.VMEM((_BPW,), jnp.float32),
            pltpu.SemaphoreType.DMA,
            pltpu.SemaphoreType.DMA,
            pltpu.SemaphoreType.DMA,
        ],
    )(_body)
    return f(xt, user, item, w_flat, bias16, ubias_flat, ibias_flat)


def kernel(mlp_concat_emebd, user, item, W, b, user_bias, item_bias):
    w_flat = W.reshape(-1)
    bias16 = jnp.broadcast_to(b.reshape(1) + _AVG, (_L,))
    return _run(mlp_concat_emebd.T, user.astype(jnp.int32),
                item.astype(jnp.int32), w_flat, bias16,
                user_bias.reshape(-1), item_bias.reshape(-1))


# speedup vs baseline: 1.0102x; 1.0102x over previous
import functools

import jax
import jax.numpy as jnp
from jax import lax
from jax.experimental import pallas as pl
from jax.experimental.pallas import tpu as pltpu
from jax.experimental.pallas import tpu_sc as plsc

_B = 16384
_D = 64
_NC = 2
_NS = 16
_NW = _NC * _NS
_BPW = _B // _NW
_AVG = 3.5
_L = 16
_DH = _D // 2
_NACC = 8


def _body(xtw_hbm, user_hbm, item_hbm, w_hbm, bias16_hbm, ubias_hbm,
          ibias_hbm, out_hbm, uidx_v, iidx_v, ub_v, ib_v, x_v, w_v, b16_v,
          out_v, gsem, xsem0, xsem1):
    wid = lax.axis_index("s") * _NC + lax.axis_index("c")
    base = wid * _BPW

    x0 = pltpu.async_copy(xtw_hbm.at[wid, pl.ds(0, _DH)],
                          x_v.at[pl.ds(0, _DH)], xsem0)

    pltpu.sync_copy(user_hbm.at[pl.ds(base, _BPW)], uidx_v)
    pltpu.sync_copy(item_hbm.at[pl.ds(base, _BPW)], iidx_v)
    ug = pltpu.async_copy(ubias_hbm.at[uidx_v], ub_v, gsem)
    ig = pltpu.async_copy(ibias_hbm.at[iidx_v], ib_v, gsem)

    x1 = pltpu.async_copy(xtw_hbm.at[wid, pl.ds(_DH, _DH)],
                          x_v.at[pl.ds(_DH, _DH)], xsem1)

    pltpu.sync_copy(w_hbm, w_v)
    pltpu.sync_copy(bias16_hbm, b16_v)

    wv = [w_v[pl.ds(k * _L, _L)] for k in range(_D // _L)]
    bconst = b16_v[pl.ds(0, _L)]

    ug.wait()
    ig.wait()

    def make_pass(j_lo, j_hi, first):
        def group(g, _):
            c = g * _L
            accs = [None] * _NACC
            for j in range(j_lo, j_hi):
                term = x_v[j, pl.ds(c, _L)] * wv[j // _L][j % _L]
                k = j % _NACC
                accs[k] = term if accs[k] is None else accs[k] + term
            if first:
                extra = bconst + ub_v[pl.ds(c, _L)] + ib_v[pl.ds(c, _L)]
            else:
                extra = out_v[pl.ds(c, _L)]
            accs[0] = accs[0] + extra
            while len(accs) > 1:
                accs = [a + b for a, b in zip(accs[::2], accs[1::2])]
            out_v[pl.ds(c, _L)] = accs[0]
            return ()
        return group

    x0.wait()
    lax.fori_loop(0, _BPW // _L, make_pass(0, _DH, True), ())
    x1.wait()
    lax.fori_loop(0, _BPW // _L, make_pass(_DH, _D, False), ())

    pltpu.sync_copy(out_v, out_hbm.at[pl.ds(base, _BPW)])


@jax.jit
def _run(xtw, user, item, w_flat, bias16, ubias_flat, ibias_flat):
    mesh = plsc.VectorSubcoreMesh(core_axis_name="c", subcore_axis_name="s")
    f = functools.partial(
        pl.kernel,
        out_type=jax.ShapeDtypeStruct((_B,), jnp.float32),
        mesh=mesh,
        scratch_types=[
            pltpu.VMEM((_BPW,), jnp.int32),
            pltpu.VMEM((_BPW,), jnp.int32),
            pltpu.VMEM((_BPW,), jnp.float32),
            pltpu.VMEM((_BPW,), jnp.float32),
            pltpu.VMEM((_D, _BPW), jnp.float32),
            pltpu.VMEM((_D,), jnp.float32),
            pltpu.VMEM((_L,), jnp.float32),
            pltpu.VMEM((_BPW,), jnp.float32),
            pltpu.SemaphoreType.DMA,
            pltpu.SemaphoreType.DMA,
            pltpu.SemaphoreType.DMA,
        ],
    )(_body)
    return f(xtw, user, item, w_flat, bias16, ubias_flat, ibias_flat)


def kernel(mlp_concat_emebd, user, item, W, b, user_bias, item_bias):
    w_flat = W.reshape(-1)
    bias16 = jnp.broadcast_to(b.reshape(1) + _AVG, (_L,))
    xtw = mlp_concat_emebd.reshape(_NW, _BPW, _D).transpose(0, 2, 1)
    return _run(xtw, user.astype(jnp.int32),
                item.astype(jnp.int32), w_flat, bias16,
                user_bias.reshape(-1), item_bias.reshape(-1))


# speedup vs baseline: 1.0329x; 1.0225x over previous
import functools

import jax
import jax.numpy as jnp
from jax import lax
from jax.experimental import pallas as pl
from jax.experimental.pallas import tpu as pltpu
from jax.experimental.pallas import tpu_sc as plsc

_B = 16384
_D = 64
_NC = 2
_NS = 16
_NW = _NC * _NS
_BPW = _B // _NW
_AVG = 3.5
_L = 16
_HALF = _BPW // 2
_NACC = 8


def _body(xt_hbm, user_hbm, item_hbm, wb_hbm, ubias_hbm, ibias_hbm,
          out_hbm, uidx_v, iidx_v, ub_v, ib_v, x_v, wb_v, out_v,
          gsem, xsem0, xsem1):
    wid = lax.axis_index("s") * _NC + lax.axis_index("c")
    base = wid * _BPW

    x0 = pltpu.async_copy(xt_hbm.at[:, pl.ds(base, _HALF)],
                          x_v.at[:, pl.ds(0, _HALF)], xsem0)

    ui = pltpu.async_copy(user_hbm.at[pl.ds(base, _BPW)], uidx_v, gsem)
    ii = pltpu.async_copy(item_hbm.at[pl.ds(base, _BPW)], iidx_v, gsem)
    ui.wait()
    ii.wait()
    ug = pltpu.async_copy(ubias_hbm.at[uidx_v], ub_v, gsem)
    ig = pltpu.async_copy(ibias_hbm.at[iidx_v], ib_v, gsem)

    x1 = pltpu.async_copy(xt_hbm.at[:, pl.ds(base + _HALF, _HALF)],
                          x_v.at[:, pl.ds(_HALF, _HALF)], xsem1)

    pltpu.sync_copy(wb_hbm, wb_v)

    wv = [wb_v[pl.ds(k * _L, _L)] for k in range(_D // _L)]
    bconst = wb_v[pl.ds(_D, _L)]

    def dot_group(g, _):
        c = g * _L
        accs = [None] * _NACC
        for j in range(_D):
            term = x_v[j, pl.ds(c, _L)] * wv[j // _L][j % _L]
            k = j % _NACC
            accs[k] = term if accs[k] is None else accs[k] + term
        accs[0] = accs[0] + bconst
        while len(accs) > 1:
            accs = [a + b for a, b in zip(accs[::2], accs[1::2])]
        out_v[pl.ds(c, _L)] = accs[0]
        return ()

    x0.wait()
    lax.fori_loop(0, _HALF // _L, dot_group, ())
    x1.wait()
    lax.fori_loop(_HALF // _L, _BPW // _L, dot_group, ())

    ug.wait()
    ig.wait()

    def bias_group(g, _):
        c = g * _L
        out_v[pl.ds(c, _L)] = (out_v[pl.ds(c, _L)]
                               + ub_v[pl.ds(c, _L)] + ib_v[pl.ds(c, _L)])
        return ()

    lax.fori_loop(0, _BPW // _L, bias_group, ())

    pltpu.sync_copy(out_v, out_hbm.at[pl.ds(base, _BPW)])


@jax.jit
def _run(xt, user, item, wb, ubias_flat, ibias_flat):
    mesh = plsc.VectorSubcoreMesh(core_axis_name="c", subcore_axis_name="s")
    f = functools.partial(
        pl.kernel,
        out_type=jax.ShapeDtypeStruct((_B,), jnp.float32),
        mesh=mesh,
        scratch_types=[
            pltpu.VMEM((_BPW,), jnp.int32),
            pltpu.VMEM((_BPW,), jnp.int32),
            pltpu.VMEM((_BPW,), jnp.float32),
            pltpu.VMEM((_BPW,), jnp.float32),
            pltpu.VMEM((_D, _BPW), jnp.float32),
            pltpu.VMEM((_D + _L,), jnp.float32),
            pltpu.VMEM((_BPW,), jnp.float32),
            pltpu.SemaphoreType.DMA,
            pltpu.SemaphoreType.DMA,
            pltpu.SemaphoreType.DMA,
        ],
    )(_body)
    return f(xt, user, item, wb, ubias_flat, ibias_flat)


def kernel(mlp_concat_emebd, user, item, W, b, user_bias, item_bias):
    wb = jnp.concatenate(
        [W.reshape(-1), jnp.broadcast_to(b.reshape(1) + _AVG, (_L,))])
    return _run(mlp_concat_emebd.T, user.astype(jnp.int32),
                item.astype(jnp.int32), wb,
                user_bias.reshape(-1), item_bias.reshape(-1))


# speedup vs baseline: 1.0606x; 1.0268x over previous
import functools

import jax
import jax.numpy as jnp
from jax import lax
from jax.experimental import pallas as pl
from jax.experimental.pallas import tpu as pltpu
from jax.experimental.pallas import tpu_sc as plsc

_B = 16384
_D = 64
_NC = 2
_NS = 16
_NW = _NC * _NS
_BPW = _B // _NW
_AVG = 3.5
_L = 16
_DH = _D // 2
_NG = _BPW // _L


def _body(xt_hbm, user_hbm, item_hbm, wb_hbm, ubias_hbm, ibias_hbm,
          out_hbm, uidx_v, iidx_v, ub_v, ib_v, x_v, wb_v, wsp_v, out_v,
          gsem, xsem0, xsem1):
    wid = lax.axis_index("s") * _NC + lax.axis_index("c")
    base = wid * _BPW

    x0 = pltpu.async_copy(xt_hbm.at[pl.ds(0, _DH), pl.ds(base, _BPW)],
                          x_v.at[pl.ds(0, _DH)], xsem0)

    ui = pltpu.async_copy(user_hbm.at[pl.ds(base, _BPW)], uidx_v, gsem)
    ii = pltpu.async_copy(item_hbm.at[pl.ds(base, _BPW)], iidx_v, gsem)
    ui.wait()
    ii.wait()
    ug = pltpu.async_copy(ubias_hbm.at[uidx_v], ub_v, gsem)
    ig = pltpu.async_copy(ibias_hbm.at[iidx_v], ib_v, gsem)

    x1 = pltpu.async_copy(xt_hbm.at[pl.ds(_DH, _DH), pl.ds(base, _BPW)],
                          x_v.at[pl.ds(_DH, _DH)], xsem1)

    pltpu.sync_copy(wb_hbm, wb_v)

    wv = [wb_v[pl.ds(k * _L, _L)] for k in range(_D // _L)]
    for j in range(_D):
        wsp_v[pl.ds(j * _L, _L)] = jnp.broadcast_to(wv[j // _L][j % _L],
                                                    (_L,))
    bconst = wb_v[pl.ds(_D, _L)]

    def feat(j, accs):
        wj = wsp_v[pl.ds(j * _L, _L)]
        return tuple(accs[g] + x_v[j, pl.ds(g * _L, _L)] * wj
                     for g in range(_NG))

    init = tuple(bconst for _ in range(_NG))
    x0.wait()
    accs = lax.fori_loop(0, _DH, feat, init)
    x1.wait()
    accs = lax.fori_loop(_DH, _D, feat, accs)

    ug.wait()
    ig.wait()

    for g in range(_NG):
        c = g * _L
        out_v[pl.ds(c, _L)] = accs[g] + (ub_v[pl.ds(c, _L)]
                                         + ib_v[pl.ds(c, _L)])

    pltpu.sync_copy(out_v, out_hbm.at[pl.ds(base, _BPW)])


@jax.jit
def _run(xt, user, item, wb, ubias_flat, ibias_flat):
    mesh = plsc.VectorSubcoreMesh(core_axis_name="c", subcore_axis_name="s")
    f = functools.partial(
        pl.kernel,
        out_type=jax.ShapeDtypeStruct((_B,), jnp.float32),
        mesh=mesh,
        scratch_types=[
            pltpu.VMEM((_BPW,), jnp.int32),
            pltpu.VMEM((_BPW,), jnp.int32),
            pltpu.VMEM((_BPW,), jnp.float32),
            pltpu.VMEM((_BPW,), jnp.float32),
            pltpu.VMEM((_D, _BPW), jnp.float32),
            pltpu.VMEM((_D + _L,), jnp.float32),
            pltpu.VMEM((_D * _L,), jnp.float32),
            pltpu.VMEM((_BPW,), jnp.float32),
            pltpu.SemaphoreType.DMA,
            pltpu.SemaphoreType.DMA,
            pltpu.SemaphoreType.DMA,
        ],
    )(_body)
    return f(xt, user, item, wb, ubias_flat, ibias_flat)


def kernel(mlp_concat_emebd, user, item, W, b, user_bias, item_bias):
    wb = jnp.concatenate(
        [W.reshape(-1), jnp.broadcast_to(b.reshape(1) + _AVG, (_L,))])
    return _run(mlp_concat_emebd.T, user.astype(jnp.int32),
                item.astype(jnp.int32), wb,
                user_bias.reshape(-1), item_bias.reshape(-1))
